# baseline (speedup 1.0000x reference)
import jax
import jax.numpy as jnp
from jax import lax
from jax.experimental import pallas as pl
from jax.experimental.pallas import tpu as pltpu
from jax.experimental.pallas import tpu_sc as plsc

HIDDEN_DIM = 64
VOCAB_SIZE = 64
MEMORY_SLOTS = 4
SEQ_LEN = 48
B = 4096
VOCAB = VOCAB_SIZE + 2

NTOK = 1 + MEMORY_SLOTS
NW = 32
CB = B // NW
ROWS_W = CB * NTOK


def _sc_gather_body(embed_hbm, idx_hbm, h_hbm, idx_v, rows_v, h_v, sem):
    c = lax.axis_index("c")
    s = lax.axis_index("s")
    wid = s * 2 + c

    pltpu.sync_copy(idx_hbm.at[pl.ds(wid * ROWS_W, ROWS_W)], idx_v)
    pltpu.async_copy(embed_hbm.at[idx_v], rows_v, sem).wait()

    def body(b, _):
        for dv in range(HIDDEN_DIM // 16):
            col = pl.ds(dv * 16, 16)
            h_v[b, col] = rows_v[NTOK * b, col]
            acc = rows_v[NTOK * b + 1, col]
            for j in range(2, NTOK):
                acc = acc + rows_v[NTOK * b + j, col]
            h_v[b, pl.ds(HIDDEN_DIM + dv * 16, 16)] = acc * (1.0 / MEMORY_SLOTS)
        return _

    lax.fori_loop(0, CB, body, 0)
    pltpu.sync_copy(h_v, h_hbm.at[pl.ds(wid * CB, CB)])


def _sc_gather(embed, idx_flat):
    embed = jnp.pad(embed, ((0, 0), (0, 128 - HIDDEN_DIM)))
    mesh = plsc.VectorSubcoreMesh(core_axis_name="c", subcore_axis_name="s")
    return pl.kernel(
        _sc_gather_body,
        mesh=mesh,
        out_type=jax.ShapeDtypeStruct((B, 2 * HIDDEN_DIM), jnp.float32),
        scratch_types=[
            pltpu.VMEM((ROWS_W,), jnp.int32),
            pltpu.VMEM((ROWS_W, 128), jnp.float32),
            pltpu.VMEM((CB, 2 * HIDDEN_DIM), jnp.float32),
            pltpu.SemaphoreType.DMA,
        ],
    )(embed, idx_flat)


def _tc_mlp(h_ref, W1_ref, b1_ref, W2_ref, b2_ref, out_ref):
    h = jnp.dot(h_ref[...], W1_ref[...], preferred_element_type=jnp.float32)
    h = jnp.maximum(h + b1_ref[...], 0.0)
    out_ref[...] = jnp.dot(h, W2_ref[...],
                           preferred_element_type=jnp.float32) + b2_ref[...]


def kernel(seqs, query_tok, embed, W1, b1, W2, b2):
    mem_idx = seqs[:, SEQ_LEN - 1 - MEMORY_SLOTS: SEQ_LEN - 1]
    idx = jnp.concatenate(
        [query_tok[:, None].astype(jnp.int32), mem_idx.astype(jnp.int32)],
        axis=1).reshape(B * NTOK)

    h = _sc_gather(embed, idx)

    TILE = 1024
    return pl.pallas_call(
        _tc_mlp,
        grid=(B // TILE,),
        in_specs=[
            pl.BlockSpec((TILE, 2 * HIDDEN_DIM), lambda i: (i, 0)),
            pl.BlockSpec((2 * HIDDEN_DIM, HIDDEN_DIM), lambda i: (0, 0)),
            pl.BlockSpec((HIDDEN_DIM,), lambda i: (0,)),
            pl.BlockSpec((HIDDEN_DIM, VOCAB_SIZE), lambda i: (0, 0)),
            pl.BlockSpec((VOCAB_SIZE,), lambda i: (0,)),
        ],
        out_specs=pl.BlockSpec((TILE, VOCAB_SIZE), lambda i: (i, 0)),
        out_shape=jax.ShapeDtypeStruct((B, VOCAB_SIZE), jnp.float32),
    )(h, W1, b1, W2, b2)


# speedup vs baseline: 1.0662x; 1.0662x over previous
import jax
import jax.numpy as jnp
from jax import lax
from jax.experimental import pallas as pl
from jax.experimental.pallas import tpu as pltpu
from jax.experimental.pallas import tpu_sc as plsc

HIDDEN_DIM = 64
VOCAB_SIZE = 64
MEMORY_SLOTS = 4
SEQ_LEN = 48
B = 4096
VOCAB = VOCAB_SIZE + 2

NTOK = 1 + MEMORY_SLOTS
NW = 32
CB = B // NW
NBV = CB // 16


def _sc_gather_body(embed_hbm, idx_hbm, ht_hbm, tbl_flat, idx_v, ht_v, sem):
    c = lax.axis_index("c")
    s = lax.axis_index("s")
    wid = s * 2 + c

    cp1 = pltpu.async_copy(embed_hbm, tbl_flat, sem)
    cp2 = pltpu.async_copy(
        idx_hbm.at[:, pl.ds(wid * CB, CB)], idx_v, sem)
    cp1.wait()
    cp2.wait()

    for bv in range(NBV):
        bcol = pl.ds(bv * 16, 16)
        base = [idx_v[k, bcol] * HIDDEN_DIM for k in range(NTOK)]

        def dbody(d, _, bcol=bcol, base=base):
            q = plsc.load_gather(tbl_flat, [base[0] + d])
            m = plsc.load_gather(tbl_flat, [base[1] + d])
            for k in range(2, NTOK):
                m = m + plsc.load_gather(tbl_flat, [base[k] + d])
            ht_v[d, bcol] = q
            ht_v[HIDDEN_DIM + d, bcol] = m
            return _

        lax.fori_loop(0, HIDDEN_DIM, dbody, 0)

    pltpu.sync_copy(ht_v, ht_hbm.at[:, pl.ds(wid * CB, CB)])


def _sc_gather(embed, idx):
    mesh = plsc.VectorSubcoreMesh(core_axis_name="c", subcore_axis_name="s")
    return pl.kernel(
        _sc_gather_body,
        mesh=mesh,
        compiler_params=pltpu.CompilerParams(needs_layout_passes=False),
        out_type=jax.ShapeDtypeStruct((2 * HIDDEN_DIM, B), jnp.float32),
        scratch_types=[
            pltpu.VMEM((VOCAB * HIDDEN_DIM,), jnp.float32),
            pltpu.VMEM((NTOK, CB), jnp.int32),
            pltpu.VMEM((2 * HIDDEN_DIM, CB), jnp.float32),
            pltpu.SemaphoreType.DMA,
        ],
    )(embed.reshape(-1), idx)


def _tc_mlp(ht_ref, W1_ref, b1_ref, W2_ref, b2_ref, out_ref):
    w_q = W1_ref[0:HIDDEN_DIM, :]
    w_m = W1_ref[HIDDEN_DIM:2 * HIDDEN_DIM, :] * (1.0 / MEMORY_SLOTS)
    w_eff = jnp.concatenate([w_q, w_m], axis=0)
    h = lax.dot_general(ht_ref[...], w_eff, (((0,), (0,)), ((), ())),
                        preferred_element_type=jnp.float32)
    h = jnp.maximum(h + b1_ref[...], 0.0)
    out_ref[...] = jnp.dot(h, W2_ref[...],
                           preferred_element_type=jnp.float32) + b2_ref[...]


def kernel(seqs, query_tok, embed, W1, b1, W2, b2):
    mem_idx = seqs[:, SEQ_LEN - 1 - MEMORY_SLOTS: SEQ_LEN - 1]
    idx = jnp.concatenate(
        [query_tok[None, :].astype(jnp.int32),
         mem_idx.T.astype(jnp.int32)], axis=0)

    ht = _sc_gather(embed, idx)

    TILE = 1024
    return pl.pallas_call(
        _tc_mlp,
        grid=(B // TILE,),
        in_specs=[
            pl.BlockSpec((2 * HIDDEN_DIM, TILE), lambda i: (0, i)),
            pl.BlockSpec((2 * HIDDEN_DIM, HIDDEN_DIM), lambda i: (0, 0)),
            pl.BlockSpec((HIDDEN_DIM,), lambda i: (0,)),
            pl.BlockSpec((HIDDEN_DIM, VOCAB_SIZE), lambda i: (0, 0)),
            pl.BlockSpec((VOCAB_SIZE,), lambda i: (0,)),
        ],
        out_specs=pl.BlockSpec((TILE, VOCAB_SIZE), lambda i: (i, 0)),
        out_shape=jax.ShapeDtypeStruct((B, VOCAB_SIZE), jnp.float32),
    )(ht, W1, b1, W2, b2)


# speedup vs baseline: 1.2082x; 1.1331x over previous
import jax
import jax.numpy as jnp
from jax import lax
from jax.experimental import pallas as pl
from jax.experimental.pallas import tpu as pltpu
from jax.experimental.pallas import tpu_sc as plsc

HIDDEN_DIM = 64
VOCAB_SIZE = 64
MEMORY_SLOTS = 4
SEQ_LEN = 48
B = 4096
VOCAB = VOCAB_SIZE + 2

NTOK = 1 + MEMORY_SLOTS
NW = 32
CB = B // NW
NBV = CB // 16


def _sc_gather_body(embed_hbm, idx_hbm, ht_hbm, tbl_flat, idx_v, ht_v, sem):
    c = lax.axis_index("c")
    s = lax.axis_index("s")
    wid = s * 2 + c

    cp1 = pltpu.async_copy(embed_hbm, tbl_flat, sem)
    cp2 = pltpu.async_copy(
        idx_hbm.at[:, pl.ds(wid * CB, CB)], idx_v, sem)
    cp1.wait()
    cp2.wait()

    for bv in range(NBV):
        bcol = pl.ds(bv * 16, 16)
        base = [idx_v[k, bcol] * HIDDEN_DIM for k in range(NTOK)]

        def dbody(d, bcol=bcol, base=base):
            q = plsc.load_gather(tbl_flat, [base[0] + d])
            m = plsc.load_gather(tbl_flat, [base[1] + d])
            for k in range(2, NTOK):
                m = m + plsc.load_gather(tbl_flat, [base[k] + d])
            ht_v[d, bcol] = q
            ht_v[HIDDEN_DIM + d, bcol] = m

        plsc.parallel_loop(0, HIDDEN_DIM, 1, unroll=8)(dbody)

    pltpu.sync_copy(ht_v, ht_hbm.at[:, pl.ds(wid * CB, CB)])


def _sc_gather(embed, idx):
    mesh = plsc.VectorSubcoreMesh(core_axis_name="c", subcore_axis_name="s")
    return pl.kernel(
        _sc_gather_body,
        mesh=mesh,
        compiler_params=pltpu.CompilerParams(needs_layout_passes=False, disable_bounds_checks=True),
        out_type=jax.ShapeDtypeStruct((2 * HIDDEN_DIM, B), jnp.float32),
        scratch_types=[
            pltpu.VMEM((VOCAB * HIDDEN_DIM,), jnp.float32),
            pltpu.VMEM((NTOK, CB), jnp.int32),
            pltpu.VMEM((2 * HIDDEN_DIM, CB), jnp.float32),
            pltpu.SemaphoreType.DMA,
        ],
    )(embed.reshape(-1), idx)


def _tc_mlp(ht_ref, W1_ref, b1_ref, W2_ref, b2_ref, out_ref):
    w_q = W1_ref[0:HIDDEN_DIM, :]
    w_m = W1_ref[HIDDEN_DIM:2 * HIDDEN_DIM, :] * (1.0 / MEMORY_SLOTS)
    w_eff = jnp.concatenate([w_q, w_m], axis=0)
    h = lax.dot_general(ht_ref[...], w_eff, (((0,), (0,)), ((), ())),
                        preferred_element_type=jnp.float32)
    h = jnp.maximum(h + b1_ref[...], 0.0)
    out_ref[...] = jnp.dot(h, W2_ref[...],
                           preferred_element_type=jnp.float32) + b2_ref[...]


def kernel(seqs, query_tok, embed, W1, b1, W2, b2):
    mem_idx = seqs[:, SEQ_LEN - 1 - MEMORY_SLOTS: SEQ_LEN - 1]
    idx = jnp.concatenate(
        [query_tok[None, :].astype(jnp.int32),
         mem_idx.T.astype(jnp.int32)], axis=0)

    ht = _sc_gather(embed, idx)

    TILE = 1024
    return pl.pallas_call(
        _tc_mlp,
        grid=(B // TILE,),
        in_specs=[
            pl.BlockSpec((2 * HIDDEN_DIM, TILE), lambda i: (0, i)),
            pl.BlockSpec((2 * HIDDEN_DIM, HIDDEN_DIM), lambda i: (0, 0)),
            pl.BlockSpec((HIDDEN_DIM,), lambda i: (0,)),
            pl.BlockSpec((HIDDEN_DIM, VOCAB_SIZE), lambda i: (0, 0)),
            pl.BlockSpec((VOCAB_SIZE,), lambda i: (0,)),
        ],
        out_specs=pl.BlockSpec((TILE, VOCAB_SIZE), lambda i: (i, 0)),
        out_shape=jax.ShapeDtypeStruct((B, VOCAB_SIZE), jnp.float32),
    )(ht, W1, b1, W2, b2)


# speedup vs baseline: 1.7371x; 1.4378x over previous
import jax
import jax.numpy as jnp
from jax import lax
from jax.experimental import pallas as pl
from jax.experimental.pallas import tpu as pltpu
from jax.experimental.pallas import tpu_sc as plsc

HIDDEN_DIM = 64
VOCAB_SIZE = 64
MEMORY_SLOTS = 4
SEQ_LEN = 48
B = 4096
VOCAB = VOCAB_SIZE + 2
TBL_STRIDE = HIDDEN_DIM + 1

NTOK = 1 + MEMORY_SLOTS
NW = 32
CB = B // NW
NBV = CB // 16


def _sc_gather_body(embed_hbm, idx_hbm, ht_hbm, tbl_flat, idx_v, ht_v, sem):
    c = lax.axis_index("c")
    s = lax.axis_index("s")
    wid = s * 2 + c

    cp1 = pltpu.async_copy(embed_hbm, tbl_flat, sem)
    cp2 = pltpu.async_copy(
        idx_hbm.at[:, pl.ds(wid * CB, CB)], idx_v, sem)
    cp1.wait()
    cp2.wait()

    for bv in range(NBV):
        bcol = pl.ds(bv * 16, 16)
        base = [idx_v[k, bcol] * TBL_STRIDE for k in range(NTOK)]

        def dbody(d, bcol=bcol, base=base):
            q = plsc.load_gather(tbl_flat, [base[0] + d])
            m = plsc.load_gather(tbl_flat, [base[1] + d])
            for k in range(2, NTOK):
                m = m + plsc.load_gather(tbl_flat, [base[k] + d])
            ht_v[d, bcol] = q
            ht_v[HIDDEN_DIM + d, bcol] = m

        plsc.parallel_loop(0, HIDDEN_DIM, 1, unroll=8)(dbody)

    pltpu.sync_copy(ht_v, ht_hbm.at[:, pl.ds(wid * CB, CB)])


def _sc_gather(embed, idx):
    mesh = plsc.VectorSubcoreMesh(core_axis_name="c", subcore_axis_name="s")
    return pl.kernel(
        _sc_gather_body,
        mesh=mesh,
        compiler_params=pltpu.CompilerParams(needs_layout_passes=False, disable_bounds_checks=True),
        out_type=jax.ShapeDtypeStruct((2 * HIDDEN_DIM, B), jnp.float32),
        scratch_types=[
            pltpu.VMEM((VOCAB * TBL_STRIDE,), jnp.float32),
            pltpu.VMEM((NTOK, CB), jnp.int32),
            pltpu.VMEM((2 * HIDDEN_DIM, CB), jnp.float32),
            pltpu.SemaphoreType.DMA,
        ],
    )(jnp.pad(embed, ((0, 0), (0, 1))).reshape(-1), idx)


def _tc_mlp(ht_ref, W1_ref, b1_ref, W2_ref, b2_ref, out_ref):
    w_q = W1_ref[0:HIDDEN_DIM, :]
    w_m = W1_ref[HIDDEN_DIM:2 * HIDDEN_DIM, :] * (1.0 / MEMORY_SLOTS)
    w_eff = jnp.concatenate([w_q, w_m], axis=0)
    h = lax.dot_general(ht_ref[...], w_eff, (((0,), (0,)), ((), ())),
                        preferred_element_type=jnp.float32)
    h = jnp.maximum(h + b1_ref[...], 0.0)
    out_ref[...] = jnp.dot(h, W2_ref[...],
                           preferred_element_type=jnp.float32) + b2_ref[...]


def kernel(seqs, query_tok, embed, W1, b1, W2, b2):
    mem_idx = seqs[:, SEQ_LEN - 1 - MEMORY_SLOTS: SEQ_LEN - 1]
    idx = jnp.concatenate(
        [query_tok[None, :].astype(jnp.int32),
         mem_idx.T.astype(jnp.int32)], axis=0)

    ht = _sc_gather(embed, idx)

    TILE = 1024
    return pl.pallas_call(
        _tc_mlp,
        grid=(B // TILE,),
        in_specs=[
            pl.BlockSpec((2 * HIDDEN_DIM, TILE), lambda i: (0, i)),
            pl.BlockSpec((2 * HIDDEN_DIM, HIDDEN_DIM), lambda i: (0, 0)),
            pl.BlockSpec((HIDDEN_DIM,), lambda i: (0,)),
            pl.BlockSpec((HIDDEN_DIM, VOCAB_SIZE), lambda i: (0, 0)),
            pl.BlockSpec((VOCAB_SIZE,), lambda i: (0,)),
        ],
        out_specs=pl.BlockSpec((TILE, VOCAB_SIZE), lambda i: (i, 0)),
        out_shape=jax.ShapeDtypeStruct((B, VOCAB_SIZE), jnp.float32),
    )(ht, W1, b1, W2, b2)


# speedup vs baseline: 1.7453x; 1.0047x over previous
import jax
import jax.numpy as jnp
from jax import lax
from jax.experimental import pallas as pl
from jax.experimental.pallas import tpu as pltpu
from jax.experimental.pallas import tpu_sc as plsc

HIDDEN_DIM = 64
VOCAB_SIZE = 64
MEMORY_SLOTS = 4
SEQ_LEN = 48
B = 4096
VOCAB = VOCAB_SIZE + 2
TBL_STRIDE = HIDDEN_DIM + 1

NTOK = 1 + MEMORY_SLOTS
NW = 32
CB = B // NW
NBV = CB // 16


def _sc_gather_body(embed_hbm, idx_hbm, ht_hbm, tbl_flat, idx_v, ht_v, sem):
    c = lax.axis_index("c")
    s = lax.axis_index("s")
    wid = s * 2 + c

    cp1 = pltpu.async_copy(embed_hbm, tbl_flat, sem)
    cp2 = pltpu.async_copy(
        idx_hbm.at[:, pl.ds(wid * CB, CB)], idx_v, sem)
    cp1.wait()
    cp2.wait()

    for bv in range(NBV):
        bcol = pl.ds(bv * 16, 16)
        base = [idx_v[k, bcol] * TBL_STRIDE for k in range(NTOK)]

        def dbody(d, bcol=bcol, base=base):
            q = plsc.load_gather(tbl_flat, [base[0] + d])
            m = plsc.load_gather(tbl_flat, [base[1] + d])
            for k in range(2, NTOK):
                m = m + plsc.load_gather(tbl_flat, [base[k] + d])
            ht_v[d, bcol] = q
            ht_v[HIDDEN_DIM + d, bcol] = m

        plsc.parallel_loop(0, HIDDEN_DIM, 1, unroll=8)(dbody)

    pltpu.sync_copy(ht_v, ht_hbm.at[:, pl.ds(wid * CB, CB)])


def _sc_gather(embed, idx):
    mesh = plsc.VectorSubcoreMesh(core_axis_name="c", subcore_axis_name="s")
    return pl.kernel(
        _sc_gather_body,
        mesh=mesh,
        compiler_params=pltpu.CompilerParams(needs_layout_passes=False, disable_bounds_checks=True, skip_device_barrier=True),
        out_type=jax.ShapeDtypeStruct((2 * HIDDEN_DIM, B), jnp.float32),
        scratch_types=[
            pltpu.VMEM((VOCAB * TBL_STRIDE,), jnp.float32),
            pltpu.VMEM((NTOK, CB), jnp.int32),
            pltpu.VMEM((2 * HIDDEN_DIM, CB), jnp.float32),
            pltpu.SemaphoreType.DMA,
        ],
    )(jnp.pad(embed, ((0, 0), (0, 1))).reshape(-1), idx)


def _tc_mlp(ht_ref, W1_ref, b1_ref, W2_ref, b2_ref, out_ref):
    w_q = W1_ref[0:HIDDEN_DIM, :]
    w_m = W1_ref[HIDDEN_DIM:2 * HIDDEN_DIM, :] * (1.0 / MEMORY_SLOTS)
    w_eff = jnp.concatenate([w_q, w_m], axis=0)
    h = lax.dot_general(ht_ref[...], w_eff, (((0,), (0,)), ((), ())),
                        preferred_element_type=jnp.float32)
    h = jnp.maximum(h + b1_ref[...], 0.0)
    out_ref[...] = jnp.dot(h, W2_ref[...],
                           preferred_element_type=jnp.float32) + b2_ref[...]


def kernel(seqs, query_tok, embed, W1, b1, W2, b2):
    mem_idx = seqs[:, SEQ_LEN - 1 - MEMORY_SLOTS: SEQ_LEN - 1]
    idx = jnp.concatenate(
        [query_tok[None, :].astype(jnp.int32),
         mem_idx.T.astype(jnp.int32)], axis=0)

    ht = _sc_gather(embed, idx)

    TILE = 1024
    return pl.pallas_call(
        _tc_mlp,
        grid=(B // TILE,),
        in_specs=[
            pl.BlockSpec((2 * HIDDEN_DIM, TILE), lambda i: (0, i)),
            pl.BlockSpec((2 * HIDDEN_DIM, HIDDEN_DIM), lambda i: (0, 0)),
            pl.BlockSpec((HIDDEN_DIM,), lambda i: (0,)),
            pl.BlockSpec((HIDDEN_DIM, VOCAB_SIZE), lambda i: (0, 0)),
            pl.BlockSpec((VOCAB_SIZE,), lambda i: (0,)),
        ],
        out_specs=pl.BlockSpec((TILE, VOCAB_SIZE), lambda i: (i, 0)),
        out_shape=jax.ShapeDtypeStruct((B, VOCAB_SIZE), jnp.float32),
    )(ht, W1, b1, W2, b2)
